# baseline (speedup 1.0000x reference)
import functools

import jax
import jax.numpy as jnp
from jax import lax
from jax.experimental import pallas as pl
from jax.experimental.pallas import tpu as pltpu
from jax.experimental.pallas import tpu_sc as plsc

N_NODES = 10000
E_EDGES = 320000
DIM = 128
E_TOT = E_EDGES + N_NODES

NC = 2
NS = 16
NW = NC * NS
LANES = 16

G = 128
N_PAD = 10240
NBKT = NW
ROWS_BKT = N_PAD // NBKT
ACC_W = 144

BN = 2048

_mesh = plsc.VectorSubcoreMesh(core_axis_name="c", subcore_axis_name="s")
_sc_params = pltpu.CompilerParams(needs_layout_passes=False)


def _ceil_chunks(n_edges):
    per = G * NW
    return -(-n_edges // per)


CW_MAIN = _ceil_chunks(E_TOT)
E_PAD = CW_MAIN * G * NW
CW_K0 = _ceil_chunks(E_EDGES)
E0_PAD = CW_K0 * G * NW

SLOT_CAP = CW_MAIN * G
PART_SZ = NBKT * E_PAD + G


@jax.jit
def _sc_loop_attr_partials(dst, ea):
    @functools.partial(
        pl.kernel,
        out_type=jax.ShapeDtypeStruct((NC, NS, 2, N_PAD), jnp.float32),
        mesh=_mesh,
        compiler_params=_sc_params,
        scratch_types=[
            pltpu.VMEM((G,), jnp.int32),
            pltpu.VMEM((G,), jnp.float32),
            pltpu.VMEM((N_PAD,), jnp.float32),
            pltpu.VMEM((N_PAD,), jnp.float32),
        ],
    )
    def body(dst_h, ea_h, out_h, idst, eav, cnt, sm):
        cid = lax.axis_index("c")
        sid = lax.axis_index("s")
        wid = sid * NC + cid

        zero16 = jnp.zeros((LANES,), jnp.float32)

        @pl.loop(0, N_PAD, step=LANES)
        def _(i):
            cnt[pl.ds(i, LANES)] = zero16
            sm[pl.ds(i, LANES)] = zero16

        lane = lax.broadcasted_iota(jnp.int32, (LANES,), 0)
        ones = jnp.ones((LANES,), jnp.float32)
        base_chunk = wid * CW_K0

        @pl.loop(0, CW_K0)
        def _(c):
            ebase = (base_chunk + c) * G
            pltpu.sync_copy(dst_h.at[pl.ds(ebase, G)], idst)
            pltpu.sync_copy(ea_h.at[pl.ds(ebase, G)], eav)

            @pl.loop(0, G, step=LANES)
            def _(e0):
                egv = eav[pl.ds(e0, LANES)]
                dstg = idst[pl.ds(e0, LANES)]
                gbase = ebase + e0
                validv = (gbase + lane) < E_EDGES
                onesm = jnp.where(validv, ones, 0.0)
                egvm = jnp.where(validv, egv, 0.0)
                for j in range(LANES):
                    m = lane == j
                    plsc.addupdate_scatter(cnt, [dstg], onesm, mask=m)
                    plsc.addupdate_scatter(sm, [dstg], egvm, mask=m)

        pltpu.sync_copy(cnt, out_h.at[cid, sid, 0])
        pltpu.sync_copy(sm, out_h.at[cid, sid, 1])

    return body(dst, ea)


@jax.jit
def _sc_partition(dst):
    @functools.partial(
        pl.kernel,
        out_type=(
            jax.ShapeDtypeStruct((PART_SZ,), jnp.int32),
            jax.ShapeDtypeStruct((NC, NS, NBKT), jnp.int32),
        ),
        mesh=_mesh,
        compiler_params=_sc_params,
        scratch_types=[
            pltpu.VMEM((G,), jnp.int32),
            pltpu.VMEM((G,), jnp.int32),
            pltpu.VMEM((G,), jnp.int32),
            pltpu.VMEM((NBKT,), jnp.int32),
            pltpu.SMEM((NBKT,), jnp.int32),
        ],
    )
    def body(dst_h, eidp_h, cnt_h, idst, eidb, posb, cdump, cnt):
        cid = lax.axis_index("c")
        sid = lax.axis_index("s")
        wid = sid * NC + cid

        @pl.loop(0, NBKT)
        def _(i):
            cnt[i] = 0

        lane = lax.broadcasted_iota(jnp.int32, (LANES,), 0)
        zero16i = jnp.zeros((LANES,), jnp.int32)
        my_slot = wid * SLOT_CAP
        trash = NBKT * E_PAD

        @pl.loop(0, CW_MAIN)
        def _(c):
            ebase = (wid * CW_MAIN + c) * G
            pltpu.sync_copy(dst_h.at[pl.ds(ebase, G)], idst)

            @pl.loop(0, G, step=LANES)
            def _(e0):
                dg = idst[pl.ds(e0, LANES)]
                eidb[pl.ds(e0, LANES)] = ebase + e0 + lane
                pvec = zero16i
                for j in range(LANES):
                    vj = (ebase + e0 + j) < E_TOT
                    d_j = dg[j]
                    b_j = d_j & (NBKT - 1)
                    cj = cnt[b_j]
                    pos_j = jnp.where(vj, b_j * E_PAD + my_slot + cj,
                                      trash + j)
                    cnt[b_j] = cj + vj.astype(jnp.int32)
                    pvec = jnp.where(lane == j, pos_j, pvec)
                posb[pl.ds(e0, LANES)] = pvec

            pltpu.sync_copy(eidb, eidp_h.at[posb])

        v0 = zero16i
        v1 = zero16i
        for j in range(LANES):
            v0 = jnp.where(lane == j, cnt[j], v0)
            v1 = jnp.where(lane == j, cnt[LANES + j], v1)
        cdump[pl.ds(0, LANES)] = v0
        cdump[pl.ds(LANES, LANES)] = v1
        pltpu.sync_copy(cdump, cnt_h.at[cid, sid])

    return body(dst)


@jax.jit
def _sc_edge_pass(eid_p, counts, src, dst, ea, xl, xr, we_v, att_v):
    @functools.partial(
        pl.kernel,
        out_type=jax.ShapeDtypeStruct((NC, NS, ROWS_BKT, ACC_W), jnp.float32),
        mesh=_mesh,
        compiler_params=_sc_params,
        scratch_types=[
            pltpu.VMEM((G,), jnp.int32),
            pltpu.VMEM((G,), jnp.int32),
            pltpu.VMEM((G,), jnp.int32),
            pltpu.VMEM((G,), jnp.float32),
            pltpu.VMEM((G, DIM), jnp.float32),
            pltpu.VMEM((G, DIM), jnp.float32),
            pltpu.VMEM((G,), jnp.int32),
            pltpu.VMEM((G,), jnp.int32),
            pltpu.VMEM((G,), jnp.int32),
            pltpu.VMEM((G,), jnp.float32),
            pltpu.VMEM((DIM,), jnp.float32),
            pltpu.VMEM((DIM,), jnp.float32),
            pltpu.VMEM((NBKT,), jnp.int32),
            pltpu.VMEM((ROWS_BKT, ACC_W), jnp.float32),
            pltpu.SemaphoreType.DMA,
        ],
    )
    def body(eidp_h, cnt_h, src_h, dst_h, ea_h, xl_h, xr_h, we_h, att_h,
             out_h,
             eidbA, isrcA, idstA, eavA, xlb, xrb,
             eidbB, isrcB, idstB, eavB,
             wev, attv, cbuf, acc, sem):
        cid = lax.axis_index("c")
        sid = lax.axis_index("s")
        bkt = 2 * sid + cid
        bhi = lax.shift_right_logical(bkt, 4)
        blo = bkt & (LANES - 1)

        zero16 = jnp.zeros((LANES,), jnp.float32)

        @pl.loop(0, ROWS_BKT)
        def _(r):
            for c0 in range(0, ACC_W, LANES):
                acc[r, pl.ds(c0, LANES)] = zero16

        pltpu.sync_copy(we_h, wev)
        pltpu.sync_copy(att_h, attv)

        lane = lax.broadcasted_iota(jnp.int32, (LANES,), 0)
        col_mask = lane == 0
        wevs = [wev[pl.ds(k * LANES, LANES)] for k in range(DIM // LANES)]
        attvs = [attv[pl.ds(k * LANES, LANES)] for k in range(DIM // LANES)]

        @pl.loop(0, NW)
        def _(w):
            cw = w & 1
            sw = lax.shift_right_logical(w, 1)
            pltpu.sync_copy(cnt_h.at[cw, sw], cbuf)
            cv = cbuf[pl.ds(bhi * LANES, LANES)]
            ns = jnp.sum(jnp.where(lane == blo, cv, 0))
            nch = lax.shift_right_logical(ns + (G - 1), 7)
            slot_base = bkt * E_PAD + w * SLOT_CAP

            def sanitize(c, eidb):
                for g in range(G // LANES):
                    sl = pl.ds(g * LANES, LANES)
                    val = (c * G + g * LANES + lane) < ns
                    eidb[sl] = jnp.where(val, eidb[sl], 0)

            def compute(c, eav, idst, xlb, xrb):
                @pl.loop(0, G, step=LANES)
                def _(e0):
                    egv = eav[pl.ds(e0, LANES)]
                    dstg = idst[pl.ds(e0, LANES)]
                    dlg = lax.shift_right_logical(dstg, 5)
                    for j in range(LANES):
                        e = e0 + j
                        vj = (c * G + e) < ns
                        ea_e = egv[j]
                        acc_v = zero16
                        for k in range(DIM // LANES):
                            sl = pl.ds(k * LANES, LANES)
                            t = xlb[e, sl] + xrb[e, sl] + ea_e * wevs[k]
                            z = jnp.maximum(t, 0.2 * t)
                            acc_v = acc_v + z * attvs[k]
                        logit = jnp.sum(acc_v)
                        exv = jnp.exp(jnp.full((LANES,), logit, jnp.float32))
                        exv = jnp.where(vj, exv, zero16)
                        dl = dlg[j]
                        for k in range(DIM // LANES):
                            sl = pl.ds(k * LANES, LANES)
                            acc[dl, sl] = acc[dl, sl] + xlb[e, sl] * exv
                        dsl = pl.ds(DIM, LANES)
                        acc[dl, dsl] = acc[dl, dsl] + jnp.where(
                            col_mask, exv, zero16)

            @pl.loop(0, nch, step=2)
            def _(cA):
                cB = cA + 1
                h1 = pltpu.async_copy(
                    eidp_h.at[pl.ds(slot_base + cA * G, G)], eidbA, sem)
                h2 = pltpu.async_copy(
                    eidp_h.at[pl.ds(slot_base + cB * G, G)], eidbB, sem)
                h1.wait()
                h2.wait()
                sanitize(cA, eidbA)
                sanitize(cB, eidbB)
                ws = [pltpu.async_copy(src_h.at[eidbA], isrcA, sem),
                      pltpu.async_copy(dst_h.at[eidbA], idstA, sem),
                      pltpu.async_copy(ea_h.at[eidbA], eavA, sem),
                      pltpu.async_copy(src_h.at[eidbB], isrcB, sem),
                      pltpu.async_copy(dst_h.at[eidbB], idstB, sem),
                      pltpu.async_copy(ea_h.at[eidbB], eavB, sem)]
                for h in ws:
                    h.wait()
                g1 = pltpu.async_copy(xl_h.at[isrcA], xlb, sem)
                g2 = pltpu.async_copy(xr_h.at[idstA], xrb, sem)
                g1.wait()
                g2.wait()
                compute(cA, eavA, idstA, xlb, xrb)
                g1 = pltpu.async_copy(xl_h.at[isrcB], xlb, sem)
                g2 = pltpu.async_copy(xr_h.at[idstB], xrb, sem)
                g1.wait()
                g2.wait()
                compute(cB, eavB, idstB, xlb, xrb)

        pltpu.sync_copy(acc, out_h.at[cid, sid])

    return body(eid_p, counts, src, dst, ea, xl, xr, we_v, att_v)


def _dotT(a, w):
    return lax.dot_general(a, w, (((1,), (1,)), ((), ())),
                           precision=lax.Precision.HIGHEST)


@jax.jit
def _tc_transform1(x, wl, wr, k0):
    def body(x_b, wl_b, wr_b, k0_b, xl_o, xr_o, la_o):
        xb = x_b[...]
        xl_o[...] = _dotT(xb, wl_b[...])
        xr_o[...] = _dotT(xb, wr_b[...])
        cnt = jnp.sum(k0_b[:, :, 0, :], axis=(0, 1))
        sm = jnp.sum(k0_b[:, :, 1, :], axis=(0, 1))
        la_o[...] = (sm / jnp.maximum(cnt, 1.0))[:, None]

    grid = (N_PAD // BN,)
    return pl.pallas_call(
        body,
        grid=grid,
        in_specs=[
            pl.BlockSpec((BN, DIM), lambda i: (i, 0)),
            pl.BlockSpec((DIM, DIM), lambda i: (0, 0)),
            pl.BlockSpec((DIM, DIM), lambda i: (0, 0)),
            pl.BlockSpec((NC, NS, 2, BN), lambda i: (0, 0, 0, i)),
        ],
        out_specs=[
            pl.BlockSpec((BN, DIM), lambda i: (i, 0)),
            pl.BlockSpec((BN, DIM), lambda i: (i, 0)),
            pl.BlockSpec((BN, 1), lambda i: (i, 0)),
        ],
        out_shape=[
            jax.ShapeDtypeStruct((N_PAD, DIM), jnp.float32),
            jax.ShapeDtypeStruct((N_PAD, DIM), jnp.float32),
            jax.ShapeDtypeStruct((N_PAD, 1), jnp.float32),
        ],
    )(x, wl, wr, k0)


@jax.jit
def _tc_combine_transform(acc, b, wl_next, wr_next):
    def body(a_b, b_b, wl_b, wr_b, co_o, xl_o, xr_o):
        a = a_b[...]
        den = a[:, DIM:DIM + 1]
        comb = a[:, :DIM] / (den + 1e-16) + b_b[...]
        co_o[...] = comb
        h = jnp.maximum(comb, 0.0)
        xl_o[...] = _dotT(h, wl_b[...])
        xr_o[...] = _dotT(h, wr_b[...])

    grid = (N_PAD // BN,)
    return pl.pallas_call(
        body,
        grid=grid,
        in_specs=[
            pl.BlockSpec((BN, ACC_W), lambda i: (i, 0)),
            pl.BlockSpec((1, DIM), lambda i: (0, 0)),
            pl.BlockSpec((DIM, DIM), lambda i: (0, 0)),
            pl.BlockSpec((DIM, DIM), lambda i: (0, 0)),
        ],
        out_specs=[
            pl.BlockSpec((BN, DIM), lambda i: (i, 0)),
            pl.BlockSpec((BN, DIM), lambda i: (i, 0)),
            pl.BlockSpec((BN, DIM), lambda i: (i, 0)),
        ],
        out_shape=[
            jax.ShapeDtypeStruct((N_PAD, DIM), jnp.float32),
            jax.ShapeDtypeStruct((N_PAD, DIM), jnp.float32),
            jax.ShapeDtypeStruct((N_PAD, DIM), jnp.float32),
        ],
    )(acc, b, wl_next, wr_next)


def kernel(x, edge_index, edge_attr, Wl1, Wr1, We1, att1, b1,
           Wl2, Wr2, We2, att2, b2):
    src0 = edge_index[0]
    dst0 = edge_index[1]
    loop = jnp.arange(N_NODES, dtype=jnp.int32)

    src = jnp.concatenate(
        [src0, loop, jnp.zeros((E_PAD - E_TOT,), jnp.int32)])
    dst = jnp.concatenate(
        [dst0, loop, jnp.zeros((E_PAD - E_TOT,), jnp.int32)])
    dst_k0 = jnp.concatenate(
        [dst0, jnp.zeros((E0_PAD - E_EDGES,), jnp.int32)])
    ea0 = jnp.concatenate(
        [edge_attr[:, 0], jnp.zeros((E0_PAD - E_EDGES,), jnp.float32)])

    x_pad = jnp.pad(x, ((0, N_PAD - N_NODES), (0, 0)))

    k0 = _sc_loop_attr_partials(dst_k0, ea0)
    xl1, xr1, loop_attr = _tc_transform1(x_pad, Wl1, Wr1, k0)

    ea = jnp.concatenate(
        [edge_attr[:, 0], loop_attr[:N_NODES, 0],
         jnp.zeros((E_PAD - E_TOT,), jnp.float32)])

    eid_p, counts = _sc_partition(dst)

    we_s = jnp.stack([We1[:, 0], We2[:, 0]])
    att_s = jnp.stack([att1, att2])
    b_s = jnp.stack([b1.reshape(1, DIM), b2.reshape(1, DIM)])
    wl_s = jnp.stack([Wl2, Wl2])
    wr_s = jnp.stack([Wr2, Wr2])

    def step(carry, p):
        xl, xr = carry
        we_i, att_i, b_i, wl_i, wr_i = p
        accs = _sc_edge_pass(eid_p, counts, src, dst, ea, xl, xr, we_i, att_i)
        acc = jnp.transpose(accs, (2, 1, 0, 3)).reshape(N_PAD, ACC_W)
        comb, xl_n, xr_n = _tc_combine_transform(acc, b_i, wl_i, wr_i)
        return (xl_n, xr_n), comb

    _, outs = lax.scan(step, (xl1, xr1), (we_s, att_s, b_s, wl_s, wr_s))
    return outs[1][:N_NODES]


# speedup vs baseline: 2.6870x; 2.6870x over previous
import functools

import jax
import jax.numpy as jnp
from jax import lax
from jax.experimental import pallas as pl
from jax.experimental.pallas import tpu as pltpu
from jax.experimental.pallas import tpu_sc as plsc

N_NODES = 10000
E_EDGES = 320000
DIM = 128
E_TOT = E_EDGES + N_NODES

NC = 2
NS = 16
NW = NC * NS
LANES = 16

G = 128
N_PAD = 10240
NBKT = NW
ROWS_BKT = N_PAD // NBKT
ACC_W = 144

BN = 2048

_mesh = plsc.VectorSubcoreMesh(core_axis_name="c", subcore_axis_name="s")
_sc_params = pltpu.CompilerParams(needs_layout_passes=False)


def _ceil_chunks(n_edges):
    per = G * NW
    return -(-n_edges // per)


CW_MAIN = _ceil_chunks(E_TOT)
E_PAD = CW_MAIN * G * NW
CW_K0 = _ceil_chunks(E_EDGES)
E0_PAD = CW_K0 * G * NW

SLOT_CAP = CW_MAIN * G
PART_SZ = NBKT * E_PAD + G


@jax.jit
def _sc_loop_attr_partials(dst, ea):
    @functools.partial(
        pl.kernel,
        out_type=jax.ShapeDtypeStruct((NC, NS, 2, N_PAD), jnp.float32),
        mesh=_mesh,
        compiler_params=_sc_params,
        scratch_types=[
            pltpu.VMEM((G,), jnp.int32),
            pltpu.VMEM((G,), jnp.float32),
            pltpu.VMEM((N_PAD,), jnp.float32),
            pltpu.VMEM((N_PAD,), jnp.float32),
        ],
    )
    def body(dst_h, ea_h, out_h, idst, eav, cnt, sm):
        cid = lax.axis_index("c")
        sid = lax.axis_index("s")
        wid = sid * NC + cid

        zero16 = jnp.zeros((LANES,), jnp.float32)

        @pl.loop(0, N_PAD, step=LANES)
        def _(i):
            cnt[pl.ds(i, LANES)] = zero16
            sm[pl.ds(i, LANES)] = zero16

        lane = lax.broadcasted_iota(jnp.int32, (LANES,), 0)
        ones = jnp.ones((LANES,), jnp.float32)
        base_chunk = wid * CW_K0

        @pl.loop(0, CW_K0)
        def _(c):
            ebase = (base_chunk + c) * G
            pltpu.sync_copy(dst_h.at[pl.ds(ebase, G)], idst)
            pltpu.sync_copy(ea_h.at[pl.ds(ebase, G)], eav)

            @pl.loop(0, G, step=LANES)
            def _(e0):
                egv = eav[pl.ds(e0, LANES)]
                dstg = idst[pl.ds(e0, LANES)]
                gbase = ebase + e0
                validv = (gbase + lane) < E_EDGES
                onesm = jnp.where(validv, ones, 0.0)
                egvm = jnp.where(validv, egv, 0.0)
                for j in range(LANES):
                    m = lane == j
                    plsc.addupdate_scatter(cnt, [dstg], onesm, mask=m)
                    plsc.addupdate_scatter(sm, [dstg], egvm, mask=m)

        pltpu.sync_copy(cnt, out_h.at[cid, sid, 0])
        pltpu.sync_copy(sm, out_h.at[cid, sid, 1])

    return body(dst, ea)


@jax.jit
def _sc_partition(src, dst, ea):
    @functools.partial(
        pl.kernel,
        out_type=(
            jax.ShapeDtypeStruct((PART_SZ,), jnp.int32),
            jax.ShapeDtypeStruct((PART_SZ,), jnp.int32),
            jax.ShapeDtypeStruct((PART_SZ,), jnp.float32),
            jax.ShapeDtypeStruct((NC, NS, NBKT), jnp.int32),
        ),
        mesh=_mesh,
        compiler_params=_sc_params,
        scratch_types=[
            pltpu.VMEM((G,), jnp.int32),
            pltpu.VMEM((G,), jnp.int32),
            pltpu.VMEM((G,), jnp.float32),
            pltpu.VMEM((G,), jnp.int32),
            pltpu.VMEM((NBKT,), jnp.int32),
            pltpu.SMEM((NBKT,), jnp.int32),
        ],
    )
    def body(src_h, dst_h, ea_h, srcp_h, dstp_h, eap_h, cnt_h,
             isrc, idst, eav, posb, cdump, cnt):
        cid = lax.axis_index("c")
        sid = lax.axis_index("s")
        wid = sid * NC + cid

        @pl.loop(0, NBKT)
        def _(i):
            cnt[i] = 0

        lane = lax.broadcasted_iota(jnp.int32, (LANES,), 0)
        zero16i = jnp.zeros((LANES,), jnp.int32)
        my_slot = wid * SLOT_CAP
        trash = NBKT * E_PAD

        @pl.loop(0, CW_MAIN)
        def _(c):
            ebase = (wid * CW_MAIN + c) * G
            pltpu.sync_copy(src_h.at[pl.ds(ebase, G)], isrc)
            pltpu.sync_copy(dst_h.at[pl.ds(ebase, G)], idst)
            pltpu.sync_copy(ea_h.at[pl.ds(ebase, G)], eav)

            @pl.loop(0, G, step=LANES)
            def _(e0):
                dg = idst[pl.ds(e0, LANES)]
                pvec = zero16i
                for j in range(LANES):
                    vj = (ebase + e0 + j) < E_TOT
                    d_j = dg[j]
                    b_j = d_j & (NBKT - 1)
                    cj = cnt[b_j]
                    pos_j = jnp.where(vj, b_j * E_PAD + my_slot + cj,
                                      trash + j)
                    cnt[b_j] = cj + vj.astype(jnp.int32)
                    pvec = jnp.where(lane == j, pos_j, pvec)
                posb[pl.ds(e0, LANES)] = pvec

            pltpu.sync_copy(isrc, srcp_h.at[posb])
            pltpu.sync_copy(idst, dstp_h.at[posb])
            pltpu.sync_copy(eav, eap_h.at[posb])

        v0 = zero16i
        v1 = zero16i
        for j in range(LANES):
            v0 = jnp.where(lane == j, cnt[j], v0)
            v1 = jnp.where(lane == j, cnt[LANES + j], v1)
        cdump[pl.ds(0, LANES)] = v0
        cdump[pl.ds(LANES, LANES)] = v1
        pltpu.sync_copy(cdump, cnt_h.at[cid, sid])

    return body(src, dst, ea)


@jax.jit
def _sc_edge_pass(src_p, dst_p, ea_p, counts, xl, xr, we_v, att_v):
    @functools.partial(
        pl.kernel,
        out_type=jax.ShapeDtypeStruct((NC, NS, ROWS_BKT, ACC_W), jnp.float32),
        mesh=_mesh,
        compiler_params=_sc_params,
        scratch_types=[
            pltpu.VMEM((G,), jnp.int32),
            pltpu.VMEM((G,), jnp.int32),
            pltpu.VMEM((G,), jnp.float32),
            pltpu.VMEM((G, DIM), jnp.float32),
            pltpu.VMEM((G, DIM), jnp.float32),
            pltpu.VMEM((DIM,), jnp.float32),
            pltpu.VMEM((DIM,), jnp.float32),
            pltpu.VMEM((NBKT,), jnp.int32),
            pltpu.VMEM((ROWS_BKT, ACC_W), jnp.float32),
        ],
    )
    def body(srcp_h, dstp_h, eap_h, cnt_h, xl_h, xr_h, we_h, att_h, out_h,
             isrc, idst, eav, xlb, xrb, wev, attv, cbuf, acc):
        cid = lax.axis_index("c")
        sid = lax.axis_index("s")
        bkt = 2 * sid + cid
        bhi = lax.shift_right_logical(bkt, 4)
        blo = bkt & (LANES - 1)

        zero16 = jnp.zeros((LANES,), jnp.float32)

        @pl.loop(0, ROWS_BKT)
        def _(r):
            for c0 in range(0, ACC_W, LANES):
                acc[r, pl.ds(c0, LANES)] = zero16

        pltpu.sync_copy(we_h, wev)
        pltpu.sync_copy(att_h, attv)

        lane = lax.broadcasted_iota(jnp.int32, (LANES,), 0)
        col_mask = lane == 0
        wevs = [wev[pl.ds(k * LANES, LANES)] for k in range(DIM // LANES)]
        attvs = [attv[pl.ds(k * LANES, LANES)] for k in range(DIM // LANES)]

        @pl.loop(0, NW)
        def _(w):
            cw = w & 1
            sw = lax.shift_right_logical(w, 1)
            pltpu.sync_copy(cnt_h.at[cw, sw], cbuf)
            cv = cbuf[pl.ds(bhi * LANES, LANES)]
            ns = jnp.sum(jnp.where(lane == blo, cv, 0))
            nch = lax.shift_right_logical(ns + (G - 1), 7)
            slot_base = bkt * E_PAD + w * SLOT_CAP

            @pl.loop(0, nch)
            def _(c):
                cbase = slot_base + c * G
                pltpu.sync_copy(srcp_h.at[pl.ds(cbase, G)], isrc)
                pltpu.sync_copy(dstp_h.at[pl.ds(cbase, G)], idst)
                pltpu.sync_copy(eap_h.at[pl.ds(cbase, G)], eav)

                for g in range(G // LANES):
                    sl = pl.ds(g * LANES, LANES)
                    val = (c * G + g * LANES + lane) < ns
                    idst[sl] = jnp.where(val, idst[sl], bkt)
                    isrc[sl] = jnp.where(val, isrc[sl], 0)

                pltpu.sync_copy(xl_h.at[isrc], xlb)
                pltpu.sync_copy(xr_h.at[idst], xrb)

                @pl.loop(0, G, step=LANES)
                def _(e0):
                    egv = eav[pl.ds(e0, LANES)]
                    dstg = idst[pl.ds(e0, LANES)]
                    dlg = lax.shift_right_logical(dstg, 5)
                    for j in range(LANES):
                        e = e0 + j
                        vj = (c * G + e) < ns
                        ea_e = egv[j]
                        acc_v = zero16
                        for k in range(DIM // LANES):
                            sl = pl.ds(k * LANES, LANES)
                            t = xlb[e, sl] + xrb[e, sl] + ea_e * wevs[k]
                            z = jnp.maximum(t, 0.2 * t)
                            acc_v = acc_v + z * attvs[k]
                        logit = jnp.sum(acc_v)
                        exv = jnp.exp(jnp.full((LANES,), logit, jnp.float32))
                        exv = jnp.where(vj, exv, zero16)
                        dl = dlg[j]
                        for k in range(DIM // LANES):
                            sl = pl.ds(k * LANES, LANES)
                            acc[dl, sl] = acc[dl, sl] + xlb[e, sl] * exv
                        dsl = pl.ds(DIM, LANES)
                        acc[dl, dsl] = acc[dl, dsl] + jnp.where(
                            col_mask, exv, zero16)

        pltpu.sync_copy(acc, out_h.at[cid, sid])

    return body(src_p, dst_p, ea_p, counts, xl, xr, we_v, att_v)


def _dotT(a, w):
    return lax.dot_general(a, w, (((1,), (1,)), ((), ())),
                           precision=lax.Precision.HIGHEST)


@jax.jit
def _tc_transform1(x, wl, wr, k0):
    def body(x_b, wl_b, wr_b, k0_b, xl_o, xr_o, la_o):
        xb = x_b[...]
        xl_o[...] = _dotT(xb, wl_b[...])
        xr_o[...] = _dotT(xb, wr_b[...])
        cnt = jnp.sum(k0_b[:, :, 0, :], axis=(0, 1))
        sm = jnp.sum(k0_b[:, :, 1, :], axis=(0, 1))
        la_o[...] = (sm / jnp.maximum(cnt, 1.0))[:, None]

    grid = (N_PAD // BN,)
    return pl.pallas_call(
        body,
        grid=grid,
        in_specs=[
            pl.BlockSpec((BN, DIM), lambda i: (i, 0)),
            pl.BlockSpec((DIM, DIM), lambda i: (0, 0)),
            pl.BlockSpec((DIM, DIM), lambda i: (0, 0)),
            pl.BlockSpec((NC, NS, 2, BN), lambda i: (0, 0, 0, i)),
        ],
        out_specs=[
            pl.BlockSpec((BN, DIM), lambda i: (i, 0)),
            pl.BlockSpec((BN, DIM), lambda i: (i, 0)),
            pl.BlockSpec((BN, 1), lambda i: (i, 0)),
        ],
        out_shape=[
            jax.ShapeDtypeStruct((N_PAD, DIM), jnp.float32),
            jax.ShapeDtypeStruct((N_PAD, DIM), jnp.float32),
            jax.ShapeDtypeStruct((N_PAD, 1), jnp.float32),
        ],
    )(x, wl, wr, k0)


@jax.jit
def _tc_combine_transform(acc, b, wl_next, wr_next):
    def body(a_b, b_b, wl_b, wr_b, co_o, xl_o, xr_o):
        a = a_b[...]
        den = a[:, DIM:DIM + 1]
        comb = a[:, :DIM] / (den + 1e-16) + b_b[...]
        co_o[...] = comb
        h = jnp.maximum(comb, 0.0)
        xl_o[...] = _dotT(h, wl_b[...])
        xr_o[...] = _dotT(h, wr_b[...])

    grid = (N_PAD // BN,)
    return pl.pallas_call(
        body,
        grid=grid,
        in_specs=[
            pl.BlockSpec((BN, ACC_W), lambda i: (i, 0)),
            pl.BlockSpec((1, DIM), lambda i: (0, 0)),
            pl.BlockSpec((DIM, DIM), lambda i: (0, 0)),
            pl.BlockSpec((DIM, DIM), lambda i: (0, 0)),
        ],
        out_specs=[
            pl.BlockSpec((BN, DIM), lambda i: (i, 0)),
            pl.BlockSpec((BN, DIM), lambda i: (i, 0)),
            pl.BlockSpec((BN, DIM), lambda i: (i, 0)),
        ],
        out_shape=[
            jax.ShapeDtypeStruct((N_PAD, DIM), jnp.float32),
            jax.ShapeDtypeStruct((N_PAD, DIM), jnp.float32),
            jax.ShapeDtypeStruct((N_PAD, DIM), jnp.float32),
        ],
    )(acc, b, wl_next, wr_next)


def kernel(x, edge_index, edge_attr, Wl1, Wr1, We1, att1, b1,
           Wl2, Wr2, We2, att2, b2):
    src0 = edge_index[0]
    dst0 = edge_index[1]
    loop = jnp.arange(N_NODES, dtype=jnp.int32)

    src = jnp.concatenate(
        [src0, loop, jnp.zeros((E_PAD - E_TOT,), jnp.int32)])
    dst = jnp.concatenate(
        [dst0, loop, jnp.zeros((E_PAD - E_TOT,), jnp.int32)])
    dst_k0 = jnp.concatenate(
        [dst0, jnp.zeros((E0_PAD - E_EDGES,), jnp.int32)])
    ea0 = jnp.concatenate(
        [edge_attr[:, 0], jnp.zeros((E0_PAD - E_EDGES,), jnp.float32)])

    x_pad = jnp.pad(x, ((0, N_PAD - N_NODES), (0, 0)))

    k0 = _sc_loop_attr_partials(dst_k0, ea0)
    xl1, xr1, loop_attr = _tc_transform1(x_pad, Wl1, Wr1, k0)

    ea = jnp.concatenate(
        [edge_attr[:, 0], loop_attr[:N_NODES, 0],
         jnp.zeros((E_PAD - E_TOT,), jnp.float32)])

    src_p, dst_p, ea_p, counts = _sc_partition(src, dst, ea)

    we_s = jnp.stack([We1[:, 0], We2[:, 0]])
    att_s = jnp.stack([att1, att2])
    b_s = jnp.stack([b1.reshape(1, DIM), b2.reshape(1, DIM)])
    wl_s = jnp.stack([Wl2, Wl2])
    wr_s = jnp.stack([Wr2, Wr2])

    def step(carry, p):
        xl, xr = carry
        we_i, att_i, b_i, wl_i, wr_i = p
        accs = _sc_edge_pass(src_p, dst_p, ea_p, counts, xl, xr, we_i, att_i)
        acc = jnp.transpose(accs, (2, 1, 0, 3)).reshape(N_PAD, ACC_W)
        comb, xl_n, xr_n = _tc_combine_transform(acc, b_i, wl_i, wr_i)
        return (xl_n, xr_n), comb

    _, outs = lax.scan(step, (xl1, xr1), (we_s, att_s, b_s, wl_s, wr_s))
    return outs[1][:N_NODES]


# speedup vs baseline: 2.7129x; 1.0096x over previous
import functools

import jax
import jax.numpy as jnp
from jax import lax
from jax.experimental import pallas as pl
from jax.experimental.pallas import tpu as pltpu
from jax.experimental.pallas import tpu_sc as plsc

N_NODES = 10000
E_EDGES = 320000
DIM = 128
E_TOT = E_EDGES + N_NODES

NC = 2
NS = 16
NW = NC * NS
LANES = 16

G = 128
N_PAD = 10240
NBKT = NW
ROWS_BKT = N_PAD // NBKT
ACC_W = 144

BN = 2048

_mesh = plsc.VectorSubcoreMesh(core_axis_name="c", subcore_axis_name="s")
_sc_params = pltpu.CompilerParams(needs_layout_passes=False)


def _ceil_chunks(n_edges):
    per = G * NW
    return -(-n_edges // per)


CW_MAIN = _ceil_chunks(E_TOT)
E_PAD = CW_MAIN * G * NW
CW_K0 = _ceil_chunks(E_EDGES)
E0_PAD = CW_K0 * G * NW

SLOT_CAP = CW_MAIN * G
PART_SZ = NBKT * E_PAD + G


@jax.jit
def _sc_loop_attr_partials(dst, ea):
    @functools.partial(
        pl.kernel,
        out_type=jax.ShapeDtypeStruct((NC, NS, 2, N_PAD), jnp.float32),
        mesh=_mesh,
        compiler_params=_sc_params,
        scratch_types=[
            pltpu.VMEM((G,), jnp.int32),
            pltpu.VMEM((G,), jnp.float32),
            pltpu.VMEM((N_PAD,), jnp.float32),
            pltpu.VMEM((N_PAD,), jnp.float32),
        ],
    )
    def body(dst_h, ea_h, out_h, idst, eav, cnt, sm):
        cid = lax.axis_index("c")
        sid = lax.axis_index("s")
        wid = sid * NC + cid

        zero16 = jnp.zeros((LANES,), jnp.float32)

        @pl.loop(0, N_PAD, step=LANES)
        def _(i):
            cnt[pl.ds(i, LANES)] = zero16
            sm[pl.ds(i, LANES)] = zero16

        lane = lax.broadcasted_iota(jnp.int32, (LANES,), 0)
        ones = jnp.ones((LANES,), jnp.float32)
        base_chunk = wid * CW_K0

        @pl.loop(0, CW_K0)
        def _(c):
            ebase = (base_chunk + c) * G
            pltpu.sync_copy(dst_h.at[pl.ds(ebase, G)], idst)
            pltpu.sync_copy(ea_h.at[pl.ds(ebase, G)], eav)

            @pl.loop(0, G, step=LANES)
            def _(e0):
                egv = eav[pl.ds(e0, LANES)]
                dstg = idst[pl.ds(e0, LANES)]
                gbase = ebase + e0
                validv = (gbase + lane) < E_EDGES
                onesm = jnp.where(validv, ones, 0.0)
                egvm = jnp.where(validv, egv, 0.0)
                for j in range(LANES):
                    m = lane == j
                    plsc.addupdate_scatter(cnt, [dstg], onesm, mask=m)
                    plsc.addupdate_scatter(sm, [dstg], egvm, mask=m)

        pltpu.sync_copy(cnt, out_h.at[cid, sid, 0])
        pltpu.sync_copy(sm, out_h.at[cid, sid, 1])

    return body(dst, ea)


@jax.jit
def _sc_partition(src, dst, ea):
    @functools.partial(
        pl.kernel,
        out_type=(
            jax.ShapeDtypeStruct((PART_SZ,), jnp.int32),
            jax.ShapeDtypeStruct((PART_SZ,), jnp.int32),
            jax.ShapeDtypeStruct((PART_SZ,), jnp.float32),
            jax.ShapeDtypeStruct((NC, NS, NBKT), jnp.int32),
        ),
        mesh=_mesh,
        compiler_params=_sc_params,
        scratch_types=[
            pltpu.VMEM((G,), jnp.int32),
            pltpu.VMEM((G,), jnp.int32),
            pltpu.VMEM((G,), jnp.float32),
            pltpu.VMEM((G,), jnp.int32),
            pltpu.VMEM((NBKT,), jnp.int32),
            pltpu.SMEM((NBKT,), jnp.int32),
        ],
    )
    def body(src_h, dst_h, ea_h, srcp_h, dstp_h, eap_h, cnt_h,
             isrc, idst, eav, posb, cdump, cnt):
        cid = lax.axis_index("c")
        sid = lax.axis_index("s")
        wid = sid * NC + cid

        @pl.loop(0, NBKT)
        def _(i):
            cnt[i] = 0

        lane = lax.broadcasted_iota(jnp.int32, (LANES,), 0)
        zero16i = jnp.zeros((LANES,), jnp.int32)
        my_slot = wid * SLOT_CAP
        trash = NBKT * E_PAD

        @pl.loop(0, CW_MAIN)
        def _(c):
            ebase = (wid * CW_MAIN + c) * G
            pltpu.sync_copy(src_h.at[pl.ds(ebase, G)], isrc)
            pltpu.sync_copy(dst_h.at[pl.ds(ebase, G)], idst)
            pltpu.sync_copy(ea_h.at[pl.ds(ebase, G)], eav)

            @pl.loop(0, G, step=LANES)
            def _(e0):
                dg = idst[pl.ds(e0, LANES)]
                pvec = zero16i
                for j in range(LANES):
                    vj = (ebase + e0 + j) < E_TOT
                    d_j = dg[j]
                    b_j = d_j & (NBKT - 1)
                    cj = cnt[b_j]
                    pos_j = jnp.where(vj, b_j * E_PAD + my_slot + cj,
                                      trash + j)
                    cnt[b_j] = cj + vj.astype(jnp.int32)
                    pvec = jnp.where(lane == j, pos_j, pvec)
                posb[pl.ds(e0, LANES)] = pvec

            pltpu.sync_copy(isrc, srcp_h.at[posb])
            pltpu.sync_copy(idst, dstp_h.at[posb])
            pltpu.sync_copy(eav, eap_h.at[posb])

        v0 = zero16i
        v1 = zero16i
        for j in range(LANES):
            v0 = jnp.where(lane == j, cnt[j], v0)
            v1 = jnp.where(lane == j, cnt[LANES + j], v1)
        cdump[pl.ds(0, LANES)] = v0
        cdump[pl.ds(LANES, LANES)] = v1
        pltpu.sync_copy(cdump, cnt_h.at[cid, sid])

    return body(src, dst, ea)


@jax.jit
def _sc_edge_pass(src_p, dst_p, ea_p, counts, xl, xr, we_v, att_v):
    @functools.partial(
        pl.kernel,
        out_type=jax.ShapeDtypeStruct((NC, NS, ROWS_BKT, ACC_W), jnp.float32),
        mesh=_mesh,
        compiler_params=_sc_params,
        scratch_types=[
            pltpu.VMEM((G,), jnp.int32),
            pltpu.VMEM((G,), jnp.int32),
            pltpu.VMEM((G,), jnp.float32),
            pltpu.VMEM((G, DIM), jnp.float32),
            pltpu.VMEM((G, DIM), jnp.float32),
            pltpu.VMEM((DIM,), jnp.float32),
            pltpu.VMEM((DIM,), jnp.float32),
            pltpu.VMEM((NBKT,), jnp.int32),
            pltpu.VMEM((ROWS_BKT, ACC_W), jnp.float32),
            pltpu.SemaphoreType.DMA,
        ],
    )
    def body(srcp_h, dstp_h, eap_h, cnt_h, xl_h, xr_h, we_h, att_h, out_h,
             isrc, idst, eav, xlb, xrb, wev, attv, cbuf, acc, sem):
        cid = lax.axis_index("c")
        sid = lax.axis_index("s")
        bkt = 2 * sid + cid
        bhi = lax.shift_right_logical(bkt, 4)
        blo = bkt & (LANES - 1)

        zero16 = jnp.zeros((LANES,), jnp.float32)

        @pl.loop(0, ROWS_BKT)
        def _(r):
            for c0 in range(0, ACC_W, LANES):
                acc[r, pl.ds(c0, LANES)] = zero16

        pltpu.sync_copy(we_h, wev)
        pltpu.sync_copy(att_h, attv)

        lane = lax.broadcasted_iota(jnp.int32, (LANES,), 0)
        col_mask = lane == 0
        wevs = [wev[pl.ds(k * LANES, LANES)] for k in range(DIM // LANES)]
        attvs = [attv[pl.ds(k * LANES, LANES)] for k in range(DIM // LANES)]

        @pl.loop(0, NW)
        def _(w):
            cw = w & 1
            sw = lax.shift_right_logical(w, 1)
            pltpu.sync_copy(cnt_h.at[cw, sw], cbuf)
            cv = cbuf[pl.ds(bhi * LANES, LANES)]
            ns = jnp.sum(jnp.where(lane == blo, cv, 0))
            nch = lax.shift_right_logical(ns + (G - 1), 7)
            slot_base = bkt * E_PAD + w * SLOT_CAP

            @pl.loop(0, nch)
            def _(c):
                cbase = slot_base + c * G
                h1 = pltpu.async_copy(srcp_h.at[pl.ds(cbase, G)], isrc, sem)
                h2 = pltpu.async_copy(dstp_h.at[pl.ds(cbase, G)], idst, sem)
                h3 = pltpu.async_copy(eap_h.at[pl.ds(cbase, G)], eav, sem)
                h1.wait()
                h2.wait()
                h3.wait()

                for g in range(G // LANES):
                    sl = pl.ds(g * LANES, LANES)
                    val = (c * G + g * LANES + lane) < ns
                    idst[sl] = jnp.where(val, idst[sl], bkt)
                    isrc[sl] = jnp.where(val, isrc[sl], 0)

                g1 = pltpu.async_copy(xl_h.at[isrc], xlb, sem)
                g2 = pltpu.async_copy(xr_h.at[idst], xrb, sem)
                g1.wait()
                g2.wait()

                @pl.loop(0, G, step=LANES)
                def _(e0):
                    egv = eav[pl.ds(e0, LANES)]
                    dstg = idst[pl.ds(e0, LANES)]
                    dlg = lax.shift_right_logical(dstg, 5)
                    for j in range(LANES):
                        e = e0 + j
                        vj = (c * G + e) < ns
                        ea_e = egv[j]
                        acc_v = zero16
                        for k in range(DIM // LANES):
                            sl = pl.ds(k * LANES, LANES)
                            t = xlb[e, sl] + xrb[e, sl] + ea_e * wevs[k]
                            z = jnp.maximum(t, 0.2 * t)
                            acc_v = acc_v + z * attvs[k]
                        logit = jnp.sum(acc_v)
                        exv = jnp.exp(jnp.full((LANES,), logit, jnp.float32))
                        exv = jnp.where(vj, exv, zero16)
                        dl = dlg[j]
                        for k in range(DIM // LANES):
                            sl = pl.ds(k * LANES, LANES)
                            acc[dl, sl] = acc[dl, sl] + xlb[e, sl] * exv
                        dsl = pl.ds(DIM, LANES)
                        acc[dl, dsl] = acc[dl, dsl] + jnp.where(
                            col_mask, exv, zero16)

        pltpu.sync_copy(acc, out_h.at[cid, sid])

    return body(src_p, dst_p, ea_p, counts, xl, xr, we_v, att_v)


def _dotT(a, w):
    return lax.dot_general(a, w, (((1,), (1,)), ((), ())),
                           precision=lax.Precision.HIGHEST)


@jax.jit
def _tc_transform1(x, wl, wr, k0):
    def body(x_b, wl_b, wr_b, k0_b, xl_o, xr_o, la_o):
        xb = x_b[...]
        xl_o[...] = _dotT(xb, wl_b[...])
        xr_o[...] = _dotT(xb, wr_b[...])
        cnt = jnp.sum(k0_b[:, :, 0, :], axis=(0, 1))
        sm = jnp.sum(k0_b[:, :, 1, :], axis=(0, 1))
        la_o[...] = (sm / jnp.maximum(cnt, 1.0))[:, None]

    grid = (N_PAD // BN,)
    return pl.pallas_call(
        body,
        grid=grid,
        in_specs=[
            pl.BlockSpec((BN, DIM), lambda i: (i, 0)),
            pl.BlockSpec((DIM, DIM), lambda i: (0, 0)),
            pl.BlockSpec((DIM, DIM), lambda i: (0, 0)),
            pl.BlockSpec((NC, NS, 2, BN), lambda i: (0, 0, 0, i)),
        ],
        out_specs=[
            pl.BlockSpec((BN, DIM), lambda i: (i, 0)),
            pl.BlockSpec((BN, DIM), lambda i: (i, 0)),
            pl.BlockSpec((BN, 1), lambda i: (i, 0)),
        ],
        out_shape=[
            jax.ShapeDtypeStruct((N_PAD, DIM), jnp.float32),
            jax.ShapeDtypeStruct((N_PAD, DIM), jnp.float32),
            jax.ShapeDtypeStruct((N_PAD, 1), jnp.float32),
        ],
    )(x, wl, wr, k0)


@jax.jit
def _tc_combine_transform(acc, b, wl_next, wr_next):
    def body(a_b, b_b, wl_b, wr_b, co_o, xl_o, xr_o):
        a = a_b[...]
        den = a[:, DIM:DIM + 1]
        comb = a[:, :DIM] / (den + 1e-16) + b_b[...]
        co_o[...] = comb
        h = jnp.maximum(comb, 0.0)
        xl_o[...] = _dotT(h, wl_b[...])
        xr_o[...] = _dotT(h, wr_b[...])

    grid = (N_PAD // BN,)
    return pl.pallas_call(
        body,
        grid=grid,
        in_specs=[
            pl.BlockSpec((BN, ACC_W), lambda i: (i, 0)),
            pl.BlockSpec((1, DIM), lambda i: (0, 0)),
            pl.BlockSpec((DIM, DIM), lambda i: (0, 0)),
            pl.BlockSpec((DIM, DIM), lambda i: (0, 0)),
        ],
        out_specs=[
            pl.BlockSpec((BN, DIM), lambda i: (i, 0)),
            pl.BlockSpec((BN, DIM), lambda i: (i, 0)),
            pl.BlockSpec((BN, DIM), lambda i: (i, 0)),
        ],
        out_shape=[
            jax.ShapeDtypeStruct((N_PAD, DIM), jnp.float32),
            jax.ShapeDtypeStruct((N_PAD, DIM), jnp.float32),
            jax.ShapeDtypeStruct((N_PAD, DIM), jnp.float32),
        ],
    )(acc, b, wl_next, wr_next)


def kernel(x, edge_index, edge_attr, Wl1, Wr1, We1, att1, b1,
           Wl2, Wr2, We2, att2, b2):
    src0 = edge_index[0]
    dst0 = edge_index[1]
    loop = jnp.arange(N_NODES, dtype=jnp.int32)

    src = jnp.concatenate(
        [src0, loop, jnp.zeros((E_PAD - E_TOT,), jnp.int32)])
    dst = jnp.concatenate(
        [dst0, loop, jnp.zeros((E_PAD - E_TOT,), jnp.int32)])
    dst_k0 = jnp.concatenate(
        [dst0, jnp.zeros((E0_PAD - E_EDGES,), jnp.int32)])
    ea0 = jnp.concatenate(
        [edge_attr[:, 0], jnp.zeros((E0_PAD - E_EDGES,), jnp.float32)])

    x_pad = jnp.pad(x, ((0, N_PAD - N_NODES), (0, 0)))

    k0 = _sc_loop_attr_partials(dst_k0, ea0)
    xl1, xr1, loop_attr = _tc_transform1(x_pad, Wl1, Wr1, k0)

    ea = jnp.concatenate(
        [edge_attr[:, 0], loop_attr[:N_NODES, 0],
         jnp.zeros((E_PAD - E_TOT,), jnp.float32)])

    src_p, dst_p, ea_p, counts = _sc_partition(src, dst, ea)

    we_s = jnp.stack([We1[:, 0], We2[:, 0]])
    att_s = jnp.stack([att1, att2])
    b_s = jnp.stack([b1.reshape(1, DIM), b2.reshape(1, DIM)])
    wl_s = jnp.stack([Wl2, Wl2])
    wr_s = jnp.stack([Wr2, Wr2])

    def step(carry, p):
        xl, xr = carry
        we_i, att_i, b_i, wl_i, wr_i = p
        accs = _sc_edge_pass(src_p, dst_p, ea_p, counts, xl, xr, we_i, att_i)
        acc = jnp.transpose(accs, (2, 1, 0, 3)).reshape(N_PAD, ACC_W)
        comb, xl_n, xr_n = _tc_combine_transform(acc, b_i, wl_i, wr_i)
        return (xl_n, xr_n), comb

    _, outs = lax.scan(step, (xl1, xr1), (we_s, att_s, b_s, wl_s, wr_s))
    return outs[1][:N_NODES]
